# baseline (speedup 1.0000x reference)
import jax
import jax.numpy as jnp
from jax.experimental import pallas as pl
from jax.experimental.pallas import tpu as pltpu

N_SEQ = 4
SEQ_LEN = 1000
F_DIM = 5143 * 3
I_DIM = 300 * 3


def _body(its_ref, sf_ref, flame_ref, inner_ref, outf_ref, outi_ref):
    del its_ref, sf_ref
    outf_ref[...] = flame_ref[...].reshape(1, F_DIM)
    outi_ref[...] = inner_ref[...].reshape(1, I_DIM)


def kernel(flame_books, inner_books, idx_to_sequence, sequence, frame):
    sf = jnp.stack([jnp.asarray(sequence, jnp.int32), jnp.asarray(frame, jnp.int32)])
    its = idx_to_sequence.astype(jnp.int32)
    flame3 = flame_books.reshape(N_SEQ * SEQ_LEN, 1, F_DIM)
    inner3 = inner_books.reshape(N_SEQ * SEQ_LEN, 1, I_DIM)

    def row_map(i, its, sf):
        return (its[sf[0]] * SEQ_LEN + sf[1], 0, 0)

    grid_spec = pltpu.PrefetchScalarGridSpec(
        num_scalar_prefetch=2,
        grid=(1,),
        in_specs=[
            pl.BlockSpec((1, 1, F_DIM), row_map),
            pl.BlockSpec((1, 1, I_DIM), row_map),
        ],
        out_specs=[
            pl.BlockSpec((1, F_DIM), lambda i, its, sf: (0, 0)),
            pl.BlockSpec((1, I_DIM), lambda i, its, sf: (0, 0)),
        ],
    )
    outf, outi = pl.pallas_call(
        _body,
        grid_spec=grid_spec,
        out_shape=[
            jax.ShapeDtypeStruct((1, F_DIM), jnp.float32),
            jax.ShapeDtypeStruct((1, I_DIM), jnp.float32),
        ],
    )(its, sf, flame3, inner3)
    return jnp.concatenate(
        [outf.reshape(-1, 3), outi.reshape(-1, 3)], axis=0
    )


# speedup vs baseline: 22.6018x; 22.6018x over previous
import jax
import jax.numpy as jnp
from jax.experimental import pallas as pl
from jax.experimental.pallas import tpu as pltpu

N_SEQ = 4
SEQ_LEN = 1000
F_DIM = 5143 * 3
I_DIM = 300 * 3


def _body(its_ref, sf_ref, flame_hbm, inner_hbm, outf_hbm, outi_hbm, semf, semi):
    idx = its_ref[sf_ref[0]]
    frame = sf_ref[1]
    cf = pltpu.make_async_copy(
        flame_hbm.at[idx, pl.ds(frame, 1), :], outf_hbm, semf)
    ci = pltpu.make_async_copy(
        inner_hbm.at[idx, pl.ds(frame, 1), :], outi_hbm, semi)
    cf.start()
    ci.start()
    cf.wait()
    ci.wait()


def kernel(flame_books, inner_books, idx_to_sequence, sequence, frame):
    sf = jnp.stack([jnp.asarray(sequence, jnp.int32), jnp.asarray(frame, jnp.int32)])
    its = idx_to_sequence.astype(jnp.int32)

    grid_spec = pltpu.PrefetchScalarGridSpec(
        num_scalar_prefetch=2,
        grid=(1,),
        in_specs=[
            pl.BlockSpec(memory_space=pl.ANY),
            pl.BlockSpec(memory_space=pl.ANY),
        ],
        out_specs=[
            pl.BlockSpec(memory_space=pl.ANY),
            pl.BlockSpec(memory_space=pl.ANY),
        ],
        scratch_shapes=[pltpu.SemaphoreType.DMA, pltpu.SemaphoreType.DMA],
    )
    outf, outi = pl.pallas_call(
        _body,
        grid_spec=grid_spec,
        out_shape=[
            jax.ShapeDtypeStruct((1, F_DIM), jnp.float32),
            jax.ShapeDtypeStruct((1, I_DIM), jnp.float32),
        ],
    )(its, sf, flame_books, inner_books)
    return jnp.concatenate(
        [outf.reshape(-1, 3), outi.reshape(-1, 3)], axis=0
    )
